# Initial kernel scaffold; baseline (speedup 1.0000x reference)
import jax
import jax.numpy as jnp
from jax.experimental import pallas as pl


def kernel(user_emb, item_emb, edge_index, edge_weight):
    raise NotImplementedError("write your pallas kernel here")


# speedup vs baseline: 9.9650x; 9.9650x over previous
import jax
import jax.numpy as jnp
from jax import lax
from jax.experimental import pallas as pl
from jax.experimental.pallas import tpu as pltpu
from jax.experimental.pallas import tpu_sc as plsc

N_USER = 25000
N_ITEM = 25000
N = N_USER + N_ITEM
E = 800000
D = 64
HALF = D // 2
BETA = 0.9

NC = 2
NS = 16

C = 128
CPT = 393
CPS = 24
NROW = NS * CPT + CPS
E_PAD = NROW * C

RB = 80
NRB = N // RB
DRAIN_IT = (NRB + NS - 1) // NS


def _body(x0_hbm, dst_hbm, src_hbm, w_hbm, out_hbm, x1_hbm,
          acc, src2, dst2, w2,
          idx3, dst3, w3, rows3, sg3, ss3,
          a_buf, x_buf):
  c = lax.axis_index("c")
  s = lax.axis_index("s")
  coff = c * N

  zero16f = jnp.zeros((16,), jnp.float32)
  rows0 = rows3[0]

  def zfill(t, _):
    rows0[t // 2, pl.ds((t % 2) * 16, 16)] = zero16f
    return _
  lax.fori_loop(0, RB * 2, zfill, None)

  def zero_slice(j2, _):
    j = s + j2 * NS
    @pl.when(j < NRB)
    def _do():
      pltpu.sync_copy(rows0.at[pl.ds(0, RB), :], acc.at[pl.ds(j * RB, RB), :])
    return _
  lax.fori_loop(0, DRAIN_IT, zero_slice, None)

  plsc.subcore_barrier()

  def spmm_pass(table_hbm):
    row0 = s * CPT

    def fire(k, b):
      @pl.when(k >= 3)
      def _ws():
        pltpu.make_async_copy(rows3[b], acc.at[dst3[b]], ss3[b]).wait()
      @pl.when(k % CPS == 0)
      def _load():
        r = row0 + k
        pltpu.sync_copy(src_hbm.at[pl.ds(r, CPS), :], src2)
        pltpu.sync_copy(dst_hbm.at[pl.ds(r, CPS), :], dst2)
        pltpu.sync_copy(w_hbm.at[pl.ds(r, CPS), :], w2)
      jj = k % CPS
      for g in range(C // 16):
        sl = pl.ds(g * 16, 16)
        idx3[b][sl] = src2[jj, sl] + coff
        dst3[b][sl] = dst2[jj, sl]
        w3[b][sl] = w2[jj, sl]
      pltpu.async_copy(table_hbm.at[idx3[b]], rows3[b], sg3[b])

    def process(b):
      pltpu.make_async_copy(table_hbm.at[idx3[b]], rows3[b], sg3[b]).wait()
      rows_b, w_b = rows3[b], w3[b]

      def mul16(g, _):
        base = jnp.full((16,), g * 16, jnp.int32)
        for u in range(16):
          wsv = plsc.load_gather(w_b, [base + u])
          e = g * 16 + u
          rows_b[e, pl.ds(0, 16)] = rows_b[e, pl.ds(0, 16)] * wsv
          rows_b[e, pl.ds(16, 16)] = rows_b[e, pl.ds(16, 16)] * wsv
        return _
      lax.fori_loop(0, C // 16, mul16, None)

      pltpu.async_copy(rows_b, acc.at[dst3[b]], ss3[b], add=True)

    def ring(kk, _):
      k0 = kk * 3
      fire(k0, 0)
      @pl.when(kk > 0)
      def _pc():
        process(2)
      fire(k0 + 1, 1)
      process(0)
      fire(k0 + 2, 2)
      process(1)
      return _
    lax.fori_loop(0, CPT // 3, ring, None)

    process(2)
    for b in range(3):
      pltpu.make_async_copy(rows3[b], acc.at[dst3[b]], ss3[b]).wait()

  def drain(dest_hbm, rezero):
    if rezero:
      lax.fori_loop(0, RB * 2, zfill, None)

    def dchunk(j2, _):
      j = s + j2 * NS
      @pl.when(j < NRB)
      def _do():
        r0 = j * RB
        pltpu.sync_copy(acc.at[pl.ds(r0, RB), :], a_buf)
        pltpu.sync_copy(x0_hbm.at[pl.ds(coff + r0, RB), :], x_buf)

        def axpy(t, _):
          row = t // 2
          cb = (t % 2) * 16
          a_buf[row, pl.ds(cb, 16)] = (a_buf[row, pl.ds(cb, 16)] * BETA
                                       + x_buf[row, pl.ds(cb, 16)])
          return _
        lax.fori_loop(0, RB * 2, axpy, None)

        pltpu.sync_copy(a_buf, dest_hbm.at[pl.ds(coff + r0, RB), :])
        if rezero:
          pltpu.sync_copy(rows0.at[pl.ds(0, RB), :],
                          acc.at[pl.ds(r0, RB), :])
      return _
    lax.fori_loop(0, DRAIN_IT, dchunk, None)

  spmm_pass(x0_hbm)
  plsc.subcore_barrier()
  drain(x1_hbm, rezero=True)
  plsc.subcore_barrier()
  spmm_pass(x1_hbm)
  plsc.subcore_barrier()
  drain(out_hbm, rezero=False)


@jax.jit
def _appnp(x0, dst, src, w):
  mesh = plsc.VectorSubcoreMesh(core_axis_name="c", subcore_axis_name="s",
                                num_cores=NC, num_subcores=NS)
  out, _ = pl.kernel(
      _body,
      out_type=(jax.ShapeDtypeStruct((NC * N, HALF), jnp.float32),
                jax.ShapeDtypeStruct((NC * N, HALF), jnp.float32)),
      mesh=mesh,
      scratch_types=[
          pltpu.VMEM_SHARED((N, HALF), jnp.float32),
          pltpu.VMEM((CPS, C), jnp.int32),
          pltpu.VMEM((CPS, C), jnp.int32),
          pltpu.VMEM((CPS, C), jnp.float32),
          [pltpu.VMEM((C,), jnp.int32)] * 3,
          [pltpu.VMEM((C,), jnp.int32)] * 3,
          [pltpu.VMEM((C,), jnp.float32)] * 3,
          [pltpu.VMEM((C, HALF), jnp.float32)] * 3,
          [pltpu.SemaphoreType.DMA] * 3,
          [pltpu.SemaphoreType.DMA] * 3,
          pltpu.VMEM((RB, HALF), jnp.float32),
          pltpu.VMEM((RB, HALF), jnp.float32),
      ],
      compiler_params=pltpu.CompilerParams(use_tc_tiling_on_sc=False,
                                           needs_layout_passes=False),
  )(x0, dst, src, w)
  return out


def kernel(user_emb, item_emb, edge_index, edge_weight):
  ego = jnp.concatenate([user_emb, item_emb], axis=0)
  x0 = jnp.concatenate([ego[:, :HALF], ego[:, HALF:]], axis=0)
  pad = E_PAD - E
  dst = jnp.concatenate([edge_index[0], jnp.zeros((pad,), jnp.int32)])
  src = jnp.concatenate([edge_index[1], jnp.zeros((pad,), jnp.int32)])
  w = jnp.concatenate([edge_weight, jnp.zeros((pad,), jnp.float32)])
  out = _appnp(x0, dst.reshape(NROW, C), src.reshape(NROW, C),
               w.reshape(NROW, C))
  full = jnp.concatenate([out[:N], out[N:]], axis=1)
  return full[:N_USER], full[N_USER:]
